# Initial kernel scaffold; baseline (speedup 1.0000x reference)
import jax
import jax.numpy as jnp
from jax.experimental import pallas as pl


def kernel(inputs, x):
    raise NotImplementedError("write your pallas kernel here")


# speedup vs baseline: 6677.2057x; 6677.2057x over previous
import functools

import jax
import jax.numpy as jnp
import numpy as np
from jax import lax
from jax.experimental import pallas as pl
from jax.experimental.pallas import tpu as pltpu
from jax.experimental.pallas import tpu_sc as plsc

_BATCH = 4096
_NBINS = 256
_NQ = 8192
_XMIN = 0.0
_XMAX = 1.0
_BW = (_XMAX - _XMIN) / _NBINS
_EPS2 = float(np.spacing(np.finfo(np.float32).eps))

_NC = 2
_NS = 16
_NW = _NC * _NS
_L = 16
_ROWS_PER_W = _BATCH // _NW


def _prep_body(a_ref, t_ref, out_ref):
    a = a_ref[...]
    t = t_ref[...]
    col = lax.broadcasted_iota(jnp.int32, a.shape, 1).astype(jnp.float32)
    centers = (2.0 * col + 1.0) * (0.5 * _BW)
    xa = jnp.clip(centers + jnp.tanh(a) * (0.5 * _BW), _XMIN, _XMAX)
    xa_r = jnp.concatenate([xa[:, 1:], xa[:, -1:]], axis=1)
    t_r = jnp.concatenate([t[:, 1:], t[:, -1:]], axis=1)
    dx = xa_r - xa
    df = t_r - t
    dx0 = jnp.abs(dx) <= _EPS2
    slope = jnp.where(dx0, 0.0, df / jnp.where(dx0, 1.0, dx))
    inter = jnp.where(dx0, t, t - slope * xa)
    out_ref[...] = jnp.concatenate([xa, slope, inter], axis=1)


_PREP_ROWS = 512


def _prep(a, t):
    return pl.pallas_call(
        _prep_body,
        grid=(_BATCH // _PREP_ROWS,),
        in_specs=[
            pl.BlockSpec((_PREP_ROWS, _NBINS), lambda i: (i, 0)),
            pl.BlockSpec((_PREP_ROWS, _NBINS), lambda i: (i, 0)),
        ],
        out_specs=pl.BlockSpec((_PREP_ROWS, 3 * _NBINS), lambda i: (i, 0)),
        out_shape=jax.ShapeDtypeStruct((_BATCH, 3 * _NBINS), jnp.float32),
    )(a, t)


def _interp_body(tab_hbm, x_hbm, out_hbm, tab_v, x_v, y_v):
    wid = lax.axis_index("s") * _NC + lax.axis_index("c")
    row0 = wid * _ROWS_PER_W

    def row_body(r, carry):
        row = row0 + r
        pltpu.sync_copy(tab_hbm.at[row], tab_v)
        pltpu.sync_copy(x_hbm.at[row], x_v)

        def q_body(k, carry2):
            off = pl.multiple_of(k * _L, _L)
            xv = x_v[pl.ds(off, _L)]
            j = (xv * float(_NBINS)).astype(jnp.int32)
            th = plsc.load_gather(tab_v, [j])
            iL = jnp.where(xv >= th, j, j - 1)
            iL = jnp.minimum(jnp.maximum(iL, 0), _NBINS - 2)
            s = plsc.load_gather(tab_v, [iL + _NBINS])
            b = plsc.load_gather(tab_v, [iL + 2 * _NBINS])
            y_v[pl.ds(off, _L)] = b + s * xv
            return carry2

        lax.fori_loop(0, _NQ // _L, q_body, 0, unroll=4)
        pltpu.sync_copy(y_v, out_hbm.at[row])
        return carry

    lax.fori_loop(0, _ROWS_PER_W, row_body, 0)


@functools.partial(jax.jit, donate_argnums=())
def _interp(tables, x):
    mesh = plsc.VectorSubcoreMesh(
        core_axis_name="c", subcore_axis_name="s", num_cores=_NC, num_subcores=_NS
    )
    return pl.kernel(
        _interp_body,
        out_type=jax.ShapeDtypeStruct((_BATCH, _NQ), jnp.float32),
        mesh=mesh,
        scratch_types=[
            pltpu.VMEM((3 * _NBINS,), jnp.float32),
            pltpu.VMEM((_NQ,), jnp.float32),
            pltpu.VMEM((_NQ,), jnp.float32),
        ],
        compiler_params=pltpu.CompilerParams(needs_layout_passes=False),
    )(tables, x)


def kernel(inputs, x):
    a = inputs[..., 0]
    t = inputs[..., 1]
    tables = _prep(a, t)
    return _interp(tables, x)


# speedup vs baseline: 25943.4291x; 3.8854x over previous
import functools

import jax
import jax.numpy as jnp
import numpy as np
from jax import lax
from jax.experimental import pallas as pl
from jax.experimental.pallas import tpu as pltpu
from jax.experimental.pallas import tpu_sc as plsc

_BATCH = 4096
_NBINS = 256
_NQ = 8192
_XMIN = 0.0
_XMAX = 1.0
_BW = (_XMAX - _XMIN) / _NBINS
_EPS2 = float(np.spacing(np.finfo(np.float32).eps))

_NC = 2
_NS = 16
_NW = _NC * _NS
_L = 16
_ROWS_PER_W = _BATCH // _NW


def _prep_body(a_ref, t_ref, out_ref):
    a = a_ref[...]
    t = t_ref[...]
    col = lax.broadcasted_iota(jnp.int32, a.shape, 1).astype(jnp.float32)
    centers = (2.0 * col + 1.0) * (0.5 * _BW)
    xa = jnp.clip(centers + jnp.tanh(a) * (0.5 * _BW), _XMIN, _XMAX)
    xa_r = jnp.concatenate([xa[:, 1:], xa[:, -1:]], axis=1)
    t_r = jnp.concatenate([t[:, 1:], t[:, -1:]], axis=1)
    dx = xa_r - xa
    df = t_r - t
    dx0 = jnp.abs(dx) <= _EPS2
    slope = jnp.where(dx0, 0.0, df / jnp.where(dx0, 1.0, dx))
    inter = jnp.where(dx0, t, t - slope * xa)
    out_ref[...] = jnp.concatenate([xa, slope, inter], axis=1)


_PREP_ROWS = 512


def _prep(a, t):
    return pl.pallas_call(
        _prep_body,
        grid=(_BATCH // _PREP_ROWS,),
        in_specs=[
            pl.BlockSpec((_PREP_ROWS, _NBINS), lambda i: (i, 0)),
            pl.BlockSpec((_PREP_ROWS, _NBINS), lambda i: (i, 0)),
        ],
        out_specs=pl.BlockSpec((_PREP_ROWS, 3 * _NBINS), lambda i: (i, 0)),
        out_shape=jax.ShapeDtypeStruct((_BATCH, 3 * _NBINS), jnp.float32),
    )(a, t)


def _interp_body(tab_hbm, x_hbm, out_hbm, tab_v, x_v, y_v):
    wid = lax.axis_index("s") * _NC + lax.axis_index("c")
    row0 = wid * _ROWS_PER_W

    def row_body(r, carry):
        row = row0 + r
        pltpu.sync_copy(tab_hbm.at[row], tab_v)
        pltpu.sync_copy(x_hbm.at[row], x_v)

        @plsc.parallel_loop(0, _NQ, step=_L, unroll=8)
        def q_body(k):
            off = pl.multiple_of(k, _L)
            xv = x_v[pl.ds(off, _L)]
            j = (xv * float(_NBINS)).astype(jnp.int32)
            th = plsc.load_gather(tab_v, [j])
            iL = jnp.where(xv >= th, j, j - 1)
            iL = jnp.minimum(jnp.maximum(iL, 0), _NBINS - 2)
            s = plsc.load_gather(tab_v, [iL + _NBINS])
            b = plsc.load_gather(tab_v, [iL + 2 * _NBINS])
            y_v[pl.ds(off, _L)] = b + s * xv
        pltpu.sync_copy(y_v, out_hbm.at[row])
        return carry

    lax.fori_loop(0, _ROWS_PER_W, row_body, 0)


@functools.partial(jax.jit, donate_argnums=())
def _interp(tables, x):
    mesh = plsc.VectorSubcoreMesh(
        core_axis_name="c", subcore_axis_name="s", num_cores=_NC, num_subcores=_NS
    )
    return pl.kernel(
        _interp_body,
        out_type=jax.ShapeDtypeStruct((_BATCH, _NQ), jnp.float32),
        mesh=mesh,
        scratch_types=[
            pltpu.VMEM((3 * _NBINS,), jnp.float32),
            pltpu.VMEM((_NQ,), jnp.float32),
            pltpu.VMEM((_NQ,), jnp.float32),
        ],
        compiler_params=pltpu.CompilerParams(needs_layout_passes=False),
    )(tables, x)


def kernel(inputs, x):
    a = inputs[..., 0]
    t = inputs[..., 1]
    tables = _prep(a, t)
    return _interp(tables, x)


# speedup vs baseline: 55162.2278x; 2.1263x over previous
import functools

import jax
import jax.numpy as jnp
import numpy as np
from jax import lax
from jax.experimental import pallas as pl
from jax.experimental.pallas import tpu as pltpu
from jax.experimental.pallas import tpu_sc as plsc

_BATCH = 4096
_NBINS = 256
_NQ = 8192
_XMIN = 0.0
_XMAX = 1.0
_BW = (_XMAX - _XMIN) / _NBINS
_EPS2 = float(np.spacing(np.finfo(np.float32).eps))

_NC = 2
_NS = 16
_NW = _NC * _NS
_L = 16
_ROWS_PER_W = _BATCH // _NW


def _prep_body(a_ref, t_ref, out_ref):
    a = a_ref[...]
    t = t_ref[...]
    col = lax.broadcasted_iota(jnp.int32, a.shape, 1).astype(jnp.float32)
    centers = (2.0 * col + 1.0) * (0.5 * _BW)
    xa = jnp.clip(centers + jnp.tanh(a) * (0.5 * _BW), _XMIN, _XMAX)
    xa_r = jnp.concatenate([xa[:, 1:], xa[:, -1:]], axis=1)
    t_r = jnp.concatenate([t[:, 1:], t[:, -1:]], axis=1)
    dx = xa_r - xa
    df = t_r - t
    dx0 = jnp.abs(dx) <= _EPS2
    slope = jnp.where(dx0, 0.0, df / jnp.where(dx0, 1.0, dx))
    inter = jnp.where(dx0, t, t - slope * xa)
    th = jnp.where(col == 0.0, -1.0, jnp.where(col == float(_NBINS - 1), 2.0, xa))
    out_ref[...] = jnp.concatenate([th, slope, inter], axis=1)


_PREP_ROWS = 512


def _prep(a, t):
    return pl.pallas_call(
        _prep_body,
        grid=(_BATCH // _PREP_ROWS,),
        in_specs=[
            pl.BlockSpec((_PREP_ROWS, _NBINS), lambda i: (i, 0)),
            pl.BlockSpec((_PREP_ROWS, _NBINS), lambda i: (i, 0)),
        ],
        out_specs=pl.BlockSpec((_PREP_ROWS, 3 * _NBINS), lambda i: (i, 0)),
        out_shape=jax.ShapeDtypeStruct((_BATCH, 3 * _NBINS), jnp.float32),
    )(a, t)


def _compute_row(tab_v, x_v, y_v):
    @plsc.parallel_loop(0, _NQ, step=_L, unroll=8)
    def q_body(k):
        off = pl.multiple_of(k, _L)
        xv = x_v[pl.ds(off, _L)]
        j = (xv * float(_NBINS)).astype(jnp.int32)
        th = plsc.load_gather(tab_v, [j])
        iL = jnp.where(xv < th, j - 1, j)
        s = plsc.load_gather(tab_v, [iL + _NBINS])
        b = plsc.load_gather(tab_v, [iL + 2 * _NBINS])
        y_v[pl.ds(off, _L)] = b + s * xv


def _interp_body(tab_hbm, x_hbm, out_hbm,
                 tab0, tab1, x0, x1, y0, y1,
                 isem0, isem1, osem0, osem1):
    wid = lax.axis_index("s") * _NC + lax.axis_index("c")
    row0 = wid * _ROWS_PER_W
    tabs, xs, ys = (tab0, tab1), (x0, x1), (y0, y1)
    isems, osems = (isem0, isem1), (osem0, osem1)

    def start_in(r, b):
        pltpu.async_copy(tab_hbm.at[row0 + r], tabs[b], isems[b])
        pltpu.async_copy(x_hbm.at[row0 + r], xs[b], isems[b])

    start_in(0, 0)
    start_in(1, 1)

    @pl.loop(0, _ROWS_PER_W, step=2)
    def _row_loop(r):
        for b in range(2):
            rb = r + b
            pltpu.make_async_copy(tab_hbm.at[row0 + rb], tabs[b], isems[b]).wait()
            pltpu.make_async_copy(x_hbm.at[row0 + rb], xs[b], isems[b]).wait()

            @pl.when(rb >= 2)
            def _wait_out():
                pltpu.make_async_copy(
                    ys[b], out_hbm.at[row0 + rb - 2], osems[b]
                ).wait()

            _compute_row(tabs[b], xs[b], ys[b])
            pltpu.async_copy(ys[b], out_hbm.at[row0 + rb], osems[b])

            @pl.when(rb < _ROWS_PER_W - 2)
            def _prefetch():
                start_in(rb + 2, b)

    for b in range(2):
        pltpu.make_async_copy(
            ys[b], out_hbm.at[row0 + _ROWS_PER_W - 2 + b], osems[b]
        ).wait()


@functools.partial(jax.jit, donate_argnums=())
def _interp(tables, x):
    mesh = plsc.VectorSubcoreMesh(
        core_axis_name="c", subcore_axis_name="s", num_cores=_NC, num_subcores=_NS
    )
    return pl.kernel(
        _interp_body,
        out_type=jax.ShapeDtypeStruct((_BATCH, _NQ), jnp.float32),
        mesh=mesh,
        scratch_types=[
            pltpu.VMEM((3 * _NBINS,), jnp.float32),
            pltpu.VMEM((3 * _NBINS,), jnp.float32),
            pltpu.VMEM((_NQ,), jnp.float32),
            pltpu.VMEM((_NQ,), jnp.float32),
            pltpu.VMEM((_NQ,), jnp.float32),
            pltpu.VMEM((_NQ,), jnp.float32),
            pltpu.SemaphoreType.DMA,
            pltpu.SemaphoreType.DMA,
            pltpu.SemaphoreType.DMA,
            pltpu.SemaphoreType.DMA,
        ],
        compiler_params=pltpu.CompilerParams(needs_layout_passes=False),
    )(tables, x)


def kernel(inputs, x):
    a = inputs[..., 0]
    t = inputs[..., 1]
    tables = _prep(a, t)
    return _interp(tables, x)
